# baseline (speedup 1.0000x reference)
import functools

import jax
import jax.numpy as jnp
from jax import lax
from jax.experimental import pallas as pl
from jax.experimental.pallas import tpu as pltpu
from jax.experimental.pallas import tpu_sc as plsc

_B, _N, _DP, _DM, _K = 4, 4096, 64, 64, 16
_TD = 80
_RB = 256
_RC = 1024
_BIG = 3.0e38


def _topk_body(feat_ref, xyz_ref, xyzT_ref, w1_ref, b1_ref, wq_ref,
               idx_ref, q_ref, tbl_ref, scores):
    b = pl.program_id(0)
    xyz_blk = xyz_ref[0]
    xyzT = xyzT_ref[0]

    g = jnp.dot(xyz_blk, xyzT, preferred_element_type=jnp.float32)
    sq_all = jnp.sum(xyzT * xyzT, axis=0, keepdims=True)
    sq_blk = jnp.sum(xyz_blk * xyz_blk, axis=1, keepdims=True)
    scores[...] = (sq_blk + sq_all) - 2.0 * g

    iota_f = lax.broadcasted_iota(jnp.int32, (_RB, _N), 1).astype(jnp.float32)
    cols = []
    for _ in range(_K // 4):
        vals = scores[...]
        for _j in range(3):
            m = jnp.min(vals, axis=1, keepdims=True)
            eq = vals == m
            cols.append(jnp.min(jnp.where(eq, iota_f, _BIG),
                                axis=1, keepdims=True))
            vals = jnp.where(eq, _BIG, vals)
        m = jnp.min(vals, axis=1, keepdims=True)
        eq = vals == m
        cols.append(jnp.min(jnp.where(eq, iota_f, _BIG),
                            axis=1, keepdims=True))
        scores[...] = jnp.where(eq, _BIG, vals)
    idx_ref[0] = jnp.concatenate(cols, axis=1).astype(jnp.int32) + b * _N

    x = jnp.dot(feat_ref[0], w1_ref[...],
                preferred_element_type=jnp.float32) + b1_ref[...]
    q_ref[0] = jnp.dot(x, wq_ref[...], preferred_element_type=jnp.float32)
    tbl_ref[:, 0:_DM] = x
    tbl_ref[:, _DM:_DM + 3] = xyz_blk
    tbl_ref[:, _DM + 3:_TD] = jnp.zeros((_RB, _TD - _DM - 3), jnp.float32)


def _run_topk(features, xyz, xyzT, W1, b1, Wq):
    nb = _N // _RB
    return pl.pallas_call(
        _topk_body,
        grid=(_B, nb),
        in_specs=[
            pl.BlockSpec((1, _RB, _DP), lambda b, i: (b, i, 0)),
            pl.BlockSpec((1, _RB, 3), lambda b, i: (b, i, 0)),
            pl.BlockSpec((1, 3, _N), lambda b, i: (b, 0, 0)),
            pl.BlockSpec((_DP, _DM), lambda b, i: (0, 0)),
            pl.BlockSpec((1, _DM), lambda b, i: (0, 0)),
            pl.BlockSpec((_DM, _DM), lambda b, i: (0, 0)),
        ],
        out_specs=[
            pl.BlockSpec((1, _RB, _K), lambda b, i: (b, i, 0)),
            pl.BlockSpec((1, _RB, _DM), lambda b, i: (b, i, 0)),
            pl.BlockSpec((_RB, _TD), lambda b, i: (b * (_N // _RB) + i, 0)),
        ],
        out_shape=[
            jax.ShapeDtypeStruct((_B, _N, _K), jnp.int32),
            jax.ShapeDtypeStruct((_B, _N, _DM), jnp.float32),
            jax.ShapeDtypeStruct((_B * _N, _TD), jnp.float32),
        ],
        scratch_shapes=[pltpu.VMEM((_RB, _N), jnp.float32)],
    )(features, xyz, xyzT, W1, b1, Wq)


_CH = 128


def _make_sc_gather(n_rows, n_idx):
    info = plsc.get_sparse_core_info()
    nw = info.num_cores * info.num_subcores
    per_w = n_idx // nw
    n_ch = per_w // _CH
    mesh = plsc.VectorSubcoreMesh(core_axis_name="c", subcore_axis_name="s")

    @functools.partial(
        pl.kernel,
        out_type=jax.ShapeDtypeStruct((n_idx, _TD), jnp.float32),
        mesh=mesh,
        scratch_types=[
            pltpu.VMEM((n_ch, _CH), jnp.int32),
            pltpu.VMEM((_CH, _TD), jnp.float32),
            pltpu.VMEM((_CH, _TD), jnp.float32),
            pltpu.SemaphoreType.DMA,
            pltpu.SemaphoreType.DMA,
        ],
        compiler_params=pltpu.CompilerParams(use_tc_tiling_on_sc=False),
    )
    def gather(tbl_hbm, idx_hbm, out_hbm, idx_v, buf0, buf1, sem0, sem1):
        wid = lax.axis_index("s") * info.num_cores + lax.axis_index("c")
        base = wid * n_ch
        pltpu.sync_copy(idx_hbm.at[pl.ds(base, n_ch)], idx_v)

        def body(j, _):
            r0 = (base + 2 * j) * _CH
            c0 = pltpu.async_copy(tbl_hbm.at[idx_v.at[2 * j]], buf0, sem0)
            c1 = pltpu.async_copy(tbl_hbm.at[idx_v.at[2 * j + 1]], buf1, sem1)
            c0.wait()
            pltpu.sync_copy(buf0, out_hbm.at[pl.ds(r0, _CH)])
            c1.wait()
            pltpu.sync_copy(buf1, out_hbm.at[pl.ds(r0 + _CH, _CH)])
            return 0

        lax.fori_loop(0, n_ch // 2, body, 0)

    return gather


def _attn_body(xg_ref, q_ref, xyz_ref, feat_ref, wk_ref, wv_ref, wd1_ref,
               bd1_ref, wd2_ref, bd2_ref, w2_ref, b2_ref, out_ref):
    xg = xg_ref[...]
    xk = xg[:, 0:_DM]
    xyzg = xg[:, _DM:_DM + 3]

    kg = jnp.dot(xk, wk_ref[...], preferred_element_type=jnp.float32)
    vg = jnp.dot(xk, wv_ref[...], preferred_element_type=jnp.float32)

    xyz_blk = xyz_ref[0]
    xyz_rep = jnp.reshape(
        jnp.broadcast_to(xyz_blk[:, None, :], (_RC, _K, 3)), (_RC * _K, 3))
    rel = xyz_rep - xyzg
    h = jnp.maximum(
        jnp.dot(rel, wd1_ref[...], preferred_element_type=jnp.float32)
        + bd1_ref[...], 0.0)
    pos = jnp.dot(h, wd2_ref[...],
                  preferred_element_type=jnp.float32) + bd2_ref[...]

    ka = jnp.reshape(kg + pos, (_RC, _K, _DM))
    va = jnp.reshape(vg + pos, (_RC, _K, _DM))
    q = q_ref[0]
    logits = jnp.sum(q[:, None, :] * ka, axis=-1) * (1.0 / 8.0)
    e = jnp.exp(logits)
    w = e / jnp.sum(e, axis=1, keepdims=True)
    att = jnp.sum(w[:, :, None] * va, axis=1)

    res = jnp.dot(att, w2_ref[...],
                  preferred_element_type=jnp.float32) + b2_ref[...]
    out_ref[0] = res + feat_ref[0]


def _run_attn(xg, q, xyz, features, Wk, Wv, Wd1, bd1, Wd2, bd2, W2, b2):
    nc = _N // _RC
    return pl.pallas_call(
        _attn_body,
        grid=(_B, nc),
        in_specs=[
            pl.BlockSpec((_RC * _K, _TD),
                         lambda b, i: (b * (_N // _RC) + i, 0)),
            pl.BlockSpec((1, _RC, _DM), lambda b, i: (b, i, 0)),
            pl.BlockSpec((1, _RC, 3), lambda b, i: (b, i, 0)),
            pl.BlockSpec((1, _RC, _DP), lambda b, i: (b, i, 0)),
            pl.BlockSpec((_DM, _DM), lambda b, i: (0, 0)),
            pl.BlockSpec((_DM, _DM), lambda b, i: (0, 0)),
            pl.BlockSpec((3, _DM), lambda b, i: (0, 0)),
            pl.BlockSpec((1, _DM), lambda b, i: (0, 0)),
            pl.BlockSpec((_DM, _DM), lambda b, i: (0, 0)),
            pl.BlockSpec((1, _DM), lambda b, i: (0, 0)),
            pl.BlockSpec((_DM, _DP), lambda b, i: (0, 0)),
            pl.BlockSpec((1, _DP), lambda b, i: (0, 0)),
        ],
        out_specs=pl.BlockSpec((1, _RC, _DP), lambda b, i: (b, i, 0)),
        out_shape=jax.ShapeDtypeStruct((_B, _N, _DP), jnp.float32),
    )(xg, q, xyz, features, Wk, Wv, Wd1, bd1, Wd2, bd2, W2, b2)


def kernel(features, xyz, W1, b1, Wq, Wk, Wv, Wd1, bd1, Wd2, bd2, W2, b2):
    xyzT = jnp.transpose(xyz, (0, 2, 1))
    idx, q, tbl = _run_topk(features, xyz, xyzT, W1,
                            jnp.reshape(b1, (1, _DM)), Wq)
    n_idx = _B * _N * _K
    gather = _make_sc_gather(_B * _N, n_idx)
    xg = gather(tbl, jnp.reshape(idx, (n_idx // _CH, _CH)))
    res = _run_attn(xg, q, xyz, features,
                    Wk, Wv, Wd1, jnp.reshape(bd1, (1, _DM)), Wd2,
                    jnp.reshape(bd2, (1, _DM)), W2, jnp.reshape(b2, (1, _DP)))
    return jnp.transpose(res, (0, 2, 1))


# speedup vs baseline: 1.0380x; 1.0380x over previous
import functools

import jax
import jax.numpy as jnp
from jax import lax
from jax.experimental import pallas as pl
from jax.experimental.pallas import tpu as pltpu
from jax.experimental.pallas import tpu_sc as plsc

_B, _N, _DP, _DM, _K = 4, 4096, 64, 64, 16
_TD = 128
_RB = 512
_RC = 1024
_BIG = 3.0e38


def _topk_body(feat_ref, xyz_ref, xyzT_ref, w1_ref, b1_ref, wq_ref,
               idx_ref, q_ref, tbl_ref, scores):
    b = pl.program_id(0)
    xyz_blk = xyz_ref[0]
    xyzT = xyzT_ref[0]

    g = jnp.dot(xyz_blk, xyzT, preferred_element_type=jnp.float32)
    sq_all = jnp.sum(xyzT * xyzT, axis=0, keepdims=True)
    sq_blk = jnp.sum(xyz_blk * xyz_blk, axis=1, keepdims=True)
    scores[...] = (sq_blk + sq_all) - 2.0 * g

    iota_f = lax.broadcasted_iota(jnp.int32, (_RB, _N), 1).astype(jnp.float32)
    cols = []
    for _ in range(_K // 4):
        vals = scores[...]
        for _j in range(3):
            m = jnp.min(vals, axis=1, keepdims=True)
            eq = vals == m
            cols.append(jnp.min(jnp.where(eq, iota_f, _BIG),
                                axis=1, keepdims=True))
            vals = jnp.where(eq, _BIG, vals)
        m = jnp.min(vals, axis=1, keepdims=True)
        eq = vals == m
        cols.append(jnp.min(jnp.where(eq, iota_f, _BIG),
                            axis=1, keepdims=True))
        scores[...] = jnp.where(eq, _BIG, vals)
    idx_ref[0] = jnp.concatenate(cols, axis=1).astype(jnp.int32) + b * _N

    x = jnp.dot(feat_ref[0], w1_ref[...],
                preferred_element_type=jnp.float32) + b1_ref[...]
    q_ref[0] = jnp.dot(x, wq_ref[...], preferred_element_type=jnp.float32)
    tbl_ref[:, 0:_DM] = x
    tbl_ref[:, _DM:_DM + 3] = xyz_blk
    tbl_ref[:, _DM + 3:_TD] = jnp.zeros((_RB, _TD - _DM - 3), jnp.float32)


def _run_topk(features, xyz, xyzT, W1, b1, Wq):
    nb = _N // _RB
    return pl.pallas_call(
        _topk_body,
        grid=(_B, nb),
        in_specs=[
            pl.BlockSpec((1, _RB, _DP), lambda b, i: (b, i, 0)),
            pl.BlockSpec((1, _RB, 3), lambda b, i: (b, i, 0)),
            pl.BlockSpec((1, 3, _N), lambda b, i: (b, 0, 0)),
            pl.BlockSpec((_DP, _DM), lambda b, i: (0, 0)),
            pl.BlockSpec((1, _DM), lambda b, i: (0, 0)),
            pl.BlockSpec((_DM, _DM), lambda b, i: (0, 0)),
        ],
        out_specs=[
            pl.BlockSpec((1, _RB, _K), lambda b, i: (b, i, 0)),
            pl.BlockSpec((1, _RB, _DM), lambda b, i: (b, i, 0)),
            pl.BlockSpec((_RB, _TD), lambda b, i: (b * (_N // _RB) + i, 0)),
        ],
        out_shape=[
            jax.ShapeDtypeStruct((_B, _N, _K), jnp.int32),
            jax.ShapeDtypeStruct((_B, _N, _DM), jnp.float32),
            jax.ShapeDtypeStruct((_B * _N, _TD), jnp.float32),
        ],
        scratch_shapes=[pltpu.VMEM((_RB, _N), jnp.float32)],
    )(features, xyz, xyzT, W1, b1, Wq)


_CH = 128


def _make_sc_gather(n_rows, n_idx):
    info = plsc.get_sparse_core_info()
    nw = info.num_cores * info.num_subcores
    per_w = n_idx // nw
    n_ch = per_w // _CH
    mesh = plsc.VectorSubcoreMesh(core_axis_name="c", subcore_axis_name="s")

    @functools.partial(
        pl.kernel,
        out_type=jax.ShapeDtypeStruct((n_idx, _TD), jnp.float32),
        mesh=mesh,
        scratch_types=[
            pltpu.VMEM((n_ch, _CH), jnp.int32),
            pltpu.VMEM((_CH, _TD), jnp.float32),
            pltpu.VMEM((_CH, _TD), jnp.float32),
            pltpu.SemaphoreType.DMA,
            pltpu.SemaphoreType.DMA,
        ],
    )
    def gather(tbl_hbm, idx_hbm, out_hbm, idx_v, buf0, buf1, sem0, sem1):
        wid = lax.axis_index("s") * info.num_cores + lax.axis_index("c")
        base = wid * n_ch
        pltpu.sync_copy(idx_hbm.at[pl.ds(base, n_ch)], idx_v)

        def body(j, _):
            r0 = (base + 2 * j) * _CH
            c0 = pltpu.async_copy(tbl_hbm.at[idx_v.at[2 * j]], buf0, sem0)
            c1 = pltpu.async_copy(tbl_hbm.at[idx_v.at[2 * j + 1]], buf1, sem1)
            c0.wait()
            pltpu.sync_copy(buf0, out_hbm.at[pl.ds(r0, _CH)])
            c1.wait()
            pltpu.sync_copy(buf1, out_hbm.at[pl.ds(r0 + _CH, _CH)])
            return 0

        lax.fori_loop(0, n_ch // 2, body, 0)

    return gather


def _attn_body(xg_ref, q_ref, xyz_ref, feat_ref, wk_ref, wv_ref, wd1_ref,
               bd1_ref, wd2_ref, bd2_ref, w2_ref, b2_ref, out_ref):
    xg = xg_ref[...]
    xk = xg[:, 0:_DM]
    xyzg = xg[:, _DM:_DM + 3]

    kg = jnp.dot(xk, wk_ref[...], preferred_element_type=jnp.float32)
    vg = jnp.dot(xk, wv_ref[...], preferred_element_type=jnp.float32)

    xyz_blk = xyz_ref[0]
    xyz_rep = jnp.reshape(
        jnp.broadcast_to(xyz_blk[:, None, :], (_RC, _K, 3)), (_RC * _K, 3))
    rel = xyz_rep - xyzg
    h = jnp.maximum(
        jnp.dot(rel, wd1_ref[...], preferred_element_type=jnp.float32)
        + bd1_ref[...], 0.0)
    pos = jnp.dot(h, wd2_ref[...],
                  preferred_element_type=jnp.float32) + bd2_ref[...]

    ka = jnp.reshape(kg + pos, (_RC, _K, _DM))
    va = jnp.reshape(vg + pos, (_RC, _K, _DM))
    q = q_ref[0]
    logits = jnp.sum(q[:, None, :] * ka, axis=-1) * (1.0 / 8.0)
    e = jnp.exp(logits)
    w = e / jnp.sum(e, axis=1, keepdims=True)
    att = jnp.sum(w[:, :, None] * va, axis=1)

    res = jnp.dot(att, w2_ref[...],
                  preferred_element_type=jnp.float32) + b2_ref[...]
    out_ref[0] = res + feat_ref[0]


def _run_attn(xg, q, xyz, features, Wk, Wv, Wd1, bd1, Wd2, bd2, W2, b2):
    nc = _N // _RC
    return pl.pallas_call(
        _attn_body,
        grid=(_B, nc),
        in_specs=[
            pl.BlockSpec((_RC * _K, _TD),
                         lambda b, i: (b * (_N // _RC) + i, 0)),
            pl.BlockSpec((1, _RC, _DM), lambda b, i: (b, i, 0)),
            pl.BlockSpec((1, _RC, 3), lambda b, i: (b, i, 0)),
            pl.BlockSpec((1, _RC, _DP), lambda b, i: (b, i, 0)),
            pl.BlockSpec((_DM, _DM), lambda b, i: (0, 0)),
            pl.BlockSpec((_DM, _DM), lambda b, i: (0, 0)),
            pl.BlockSpec((3, _DM), lambda b, i: (0, 0)),
            pl.BlockSpec((1, _DM), lambda b, i: (0, 0)),
            pl.BlockSpec((_DM, _DM), lambda b, i: (0, 0)),
            pl.BlockSpec((1, _DM), lambda b, i: (0, 0)),
            pl.BlockSpec((_DM, _DP), lambda b, i: (0, 0)),
            pl.BlockSpec((1, _DP), lambda b, i: (0, 0)),
        ],
        out_specs=pl.BlockSpec((1, _RC, _DP), lambda b, i: (b, i, 0)),
        out_shape=jax.ShapeDtypeStruct((_B, _N, _DP), jnp.float32),
    )(xg, q, xyz, features, Wk, Wv, Wd1, bd1, Wd2, bd2, W2, b2)


def kernel(features, xyz, W1, b1, Wq, Wk, Wv, Wd1, bd1, Wd2, bd2, W2, b2):
    xyzT = jnp.transpose(xyz, (0, 2, 1))
    idx, q, tbl = _run_topk(features, xyz, xyzT, W1,
                            jnp.reshape(b1, (1, _DM)), Wq)
    n_idx = _B * _N * _K
    gather = _make_sc_gather(_B * _N, n_idx)
    xg = gather(tbl, jnp.reshape(idx, (n_idx // _CH, _CH)))
    res = _run_attn(xg, q, xyz, features,
                    Wk, Wv, Wd1, jnp.reshape(bd1, (1, _DM)), Wd2,
                    jnp.reshape(bd2, (1, _DM)), W2, jnp.reshape(b2, (1, _DP)))
    return jnp.transpose(res, (0, 2, 1))


# speedup vs baseline: 1.0930x; 1.0530x over previous
import functools

import jax
import jax.numpy as jnp
from jax import lax
from jax.experimental import pallas as pl
from jax.experimental.pallas import tpu as pltpu
from jax.experimental.pallas import tpu_sc as plsc

_B, _N, _DP, _DM, _K = 4, 4096, 64, 64, 16
_TD = 128
_RB = 256
_RC = 1024
_BIG = 3.0e38


def _topk_body(feat_ref, xyz_ref, xyzT_ref, w1_ref, b1_ref, wq_ref,
               idx_ref, q_ref, tbl_ref, scores):
    b = pl.program_id(0)
    xyz_blk = xyz_ref[0]
    xyzT = xyzT_ref[0]

    g = jnp.dot(xyz_blk, xyzT, preferred_element_type=jnp.float32)
    sq_all = jnp.sum(xyzT * xyzT, axis=0, keepdims=True)
    sq_blk = jnp.sum(xyz_blk * xyz_blk, axis=1, keepdims=True)
    scores[...] = (sq_blk + sq_all) - 2.0 * g

    iota_f = lax.broadcasted_iota(jnp.int32, (_RB, _N), 1).astype(jnp.float32)
    cols = []
    for _ in range(_K // 4):
        vals = scores[...]
        for _j in range(3):
            m = jnp.min(vals, axis=1, keepdims=True)
            eq = vals == m
            cols.append(jnp.min(jnp.where(eq, iota_f, _BIG),
                                axis=1, keepdims=True))
            vals = jnp.where(eq, _BIG, vals)
        m = jnp.min(vals, axis=1, keepdims=True)
        eq = vals == m
        cols.append(jnp.min(jnp.where(eq, iota_f, _BIG),
                            axis=1, keepdims=True))
        scores[...] = jnp.where(eq, _BIG, vals)
    idx_ref[0] = jnp.concatenate(cols, axis=1).astype(jnp.int32) + b * _N

    x = jnp.dot(feat_ref[0], w1_ref[...],
                preferred_element_type=jnp.float32) + b1_ref[...]
    q_ref[0] = jnp.dot(x, wq_ref[...], preferred_element_type=jnp.float32)
    tbl_ref[:, 0:_DM] = x
    tbl_ref[:, _DM:_DM + 3] = xyz_blk
    tbl_ref[:, _DM + 3:_TD] = jnp.zeros((_RB, _TD - _DM - 3), jnp.float32)


def _run_topk(features, xyz, xyzT, W1, b1, Wq):
    nb = _N // _RB
    return pl.pallas_call(
        _topk_body,
        grid=(_B, nb),
        in_specs=[
            pl.BlockSpec((1, _RB, _DP), lambda b, i: (b, i, 0)),
            pl.BlockSpec((1, _RB, 3), lambda b, i: (b, i, 0)),
            pl.BlockSpec((1, 3, _N), lambda b, i: (b, 0, 0)),
            pl.BlockSpec((_DP, _DM), lambda b, i: (0, 0)),
            pl.BlockSpec((1, _DM), lambda b, i: (0, 0)),
            pl.BlockSpec((_DM, _DM), lambda b, i: (0, 0)),
        ],
        out_specs=[
            pl.BlockSpec((1, _RB, _K), lambda b, i: (b, i, 0)),
            pl.BlockSpec((1, _RB, _DM), lambda b, i: (b, i, 0)),
            pl.BlockSpec((_RB, _TD), lambda b, i: (b * (_N // _RB) + i, 0)),
        ],
        out_shape=[
            jax.ShapeDtypeStruct((_B, _N, _K), jnp.int32),
            jax.ShapeDtypeStruct((_B, _N, _DM), jnp.float32),
            jax.ShapeDtypeStruct((_B * _N, _TD), jnp.float32),
        ],
        scratch_shapes=[pltpu.VMEM((_RB, _N), jnp.float32)],
    )(features, xyz, xyzT, W1, b1, Wq)


_CH = 128


def _make_sc_gather(n_rows, n_idx):
    info = plsc.get_sparse_core_info()
    nw = info.num_cores * info.num_subcores
    per_w = n_idx // nw
    n_ch = per_w // _CH
    mesh = plsc.VectorSubcoreMesh(core_axis_name="c", subcore_axis_name="s")

    @functools.partial(
        pl.kernel,
        out_type=jax.ShapeDtypeStruct((n_idx, _TD), jnp.float32),
        mesh=mesh,
        scratch_types=[
            pltpu.VMEM((n_ch, _CH), jnp.int32),
            pltpu.VMEM((_CH, _TD), jnp.float32),
            pltpu.VMEM((_CH, _TD), jnp.float32),
            pltpu.SemaphoreType.DMA,
            pltpu.SemaphoreType.DMA,
        ],
    )
    def gather(tbl_hbm, idx_hbm, out_hbm, idx_v, buf0, buf1, sem0, sem1):
        wid = lax.axis_index("s") * info.num_cores + lax.axis_index("c")
        base = wid * n_ch
        pltpu.sync_copy(idx_hbm.at[pl.ds(base, n_ch)], idx_v)

        def body(j, _):
            r0 = (base + 2 * j) * _CH
            c0 = pltpu.async_copy(tbl_hbm.at[idx_v.at[2 * j]], buf0, sem0)
            c1 = pltpu.async_copy(tbl_hbm.at[idx_v.at[2 * j + 1]], buf1, sem1)
            c0.wait()
            pltpu.sync_copy(buf0, out_hbm.at[pl.ds(r0, _CH)])
            c1.wait()
            pltpu.sync_copy(buf1, out_hbm.at[pl.ds(r0 + _CH, _CH)])
            return 0

        lax.fori_loop(0, n_ch // 2, body, 0)

    return gather


def _attn_body(xg_ref, q_ref, xyz_ref, feat_ref, wk_ref, wv_ref, wd1_ref,
               bd1_ref, wd2_ref, bd2_ref, w2_ref, b2_ref, out_ref):
    xg = xg_ref[...]
    xk = xg[:, 0:_DM]
    xyzg = xg[:, _DM:_DM + 3]

    kg = jnp.dot(xk, wk_ref[...], preferred_element_type=jnp.float32)
    vg = jnp.dot(xk, wv_ref[...], preferred_element_type=jnp.float32)

    xyz_blk = xyz_ref[0]
    xyz_rep = jnp.reshape(
        jnp.broadcast_to(xyz_blk[:, None, :], (_RC, _K, 3)), (_RC * _K, 3))
    rel = xyz_rep - xyzg
    h = jnp.maximum(
        jnp.dot(rel, wd1_ref[...], preferred_element_type=jnp.float32)
        + bd1_ref[...], 0.0)
    pos = jnp.dot(h, wd2_ref[...],
                  preferred_element_type=jnp.float32) + bd2_ref[...]

    ka = jnp.reshape(kg + pos, (_RC, _K, _DM))
    va = jnp.reshape(vg + pos, (_RC, _K, _DM))
    q = q_ref[0]
    logits = jnp.sum(q[:, None, :] * ka, axis=-1) * (1.0 / 8.0)
    e = jnp.exp(logits)
    w = e / jnp.sum(e, axis=1, keepdims=True)
    att = jnp.sum(w[:, :, None] * va, axis=1)

    res = jnp.dot(att, w2_ref[...],
                  preferred_element_type=jnp.float32) + b2_ref[...]
    out_ref[0] = res + feat_ref[0]


def _run_attn(xg, q, xyz, features, Wk, Wv, Wd1, bd1, Wd2, bd2, W2, b2):
    nc = _N // _RC
    return pl.pallas_call(
        _attn_body,
        grid=(_B, nc),
        in_specs=[
            pl.BlockSpec((_RC * _K, _TD),
                         lambda b, i: (b * (_N // _RC) + i, 0)),
            pl.BlockSpec((1, _RC, _DM), lambda b, i: (b, i, 0)),
            pl.BlockSpec((1, _RC, 3), lambda b, i: (b, i, 0)),
            pl.BlockSpec((1, _RC, _DP), lambda b, i: (b, i, 0)),
            pl.BlockSpec((_DM, _DM), lambda b, i: (0, 0)),
            pl.BlockSpec((_DM, _DM), lambda b, i: (0, 0)),
            pl.BlockSpec((3, _DM), lambda b, i: (0, 0)),
            pl.BlockSpec((1, _DM), lambda b, i: (0, 0)),
            pl.BlockSpec((_DM, _DM), lambda b, i: (0, 0)),
            pl.BlockSpec((1, _DM), lambda b, i: (0, 0)),
            pl.BlockSpec((_DM, _DP), lambda b, i: (0, 0)),
            pl.BlockSpec((1, _DP), lambda b, i: (0, 0)),
        ],
        out_specs=pl.BlockSpec((1, _RC, _DP), lambda b, i: (b, i, 0)),
        out_shape=jax.ShapeDtypeStruct((_B, _N, _DP), jnp.float32),
    )(xg, q, xyz, features, Wk, Wv, Wd1, bd1, Wd2, bd2, W2, b2)


def kernel(features, xyz, W1, b1, Wq, Wk, Wv, Wd1, bd1, Wd2, bd2, W2, b2):
    xyzT = jnp.transpose(xyz, (0, 2, 1))
    idx, q, tbl = _run_topk(features, xyz, xyzT, W1,
                            jnp.reshape(b1, (1, _DM)), Wq)
    n_idx = _B * _N * _K
    gather = _make_sc_gather(_B * _N, n_idx)
    xg = gather(tbl, jnp.reshape(idx, (n_idx // _CH, _CH)))
    res = _run_attn(xg, q, xyz, features,
                    Wk, Wv, Wd1, jnp.reshape(bd1, (1, _DM)), Wd2,
                    jnp.reshape(bd2, (1, _DM)), W2, jnp.reshape(b2, (1, _DP)))
    return jnp.transpose(res, (0, 2, 1))
